# baseline (speedup 1.0000x reference)
import functools

import jax
import jax.numpy as jnp
from jax import lax
from jax.experimental import pallas as pl
from jax.experimental.pallas import tpu as pltpu
from jax.experimental.pallas import tpu_sc as plsc

D = 32
B = 16384 * 26
CHUNK = 1024
NBUF = 3


def _sc_gather(table, idx_flat):
    info = plsc.get_sparse_core_info()
    nw = info.num_cores * info.num_subcores
    b_per_w = B // nw
    n_chunks = b_per_w // CHUNK
    mesh = plsc.VectorSubcoreMesh(core_axis_name="c", subcore_axis_name="s")

    @functools.partial(
        pl.kernel,
        mesh=mesh,
        out_type=jax.ShapeDtypeStruct((B, D), jnp.float32),
        scratch_types=[
            pltpu.VMEM((b_per_w,), jnp.int32),
            [pltpu.VMEM((CHUNK, D), jnp.float32) for _ in range(NBUF)],
            [pltpu.SemaphoreType.DMA for _ in range(NBUF)],
            [pltpu.SemaphoreType.DMA for _ in range(NBUF)],
        ],
        compiler_params=pltpu.CompilerParams(use_tc_tiling_on_sc=False),
    )
    def k(table_hbm, idx_hbm, out_hbm, idx_v, rows, gsem, ssem):
        wid = lax.axis_index("s") * info.num_cores + lax.axis_index("c")
        base = wid * b_per_w
        pltpu.sync_copy(idx_hbm.at[pl.ds(base, b_per_w)], idx_v)

        def gather(i, b):
            return pltpu.async_copy(
                table_hbm.at[idx_v.at[pl.ds(i * CHUNK, CHUNK)]], rows[b], gsem[b]
            )

        gath = [gather(b, b) for b in range(NBUF)]
        store = [None] * NBUF
        for i in range(n_chunks):
            b = i % NBUF
            gath[b].wait()
            store[b] = pltpu.async_copy(
                rows[b], out_hbm.at[pl.ds(base + i * CHUNK, CHUNK)], ssem[b]
            )
            nxt = i + NBUF
            if nxt < n_chunks:
                store[b].wait()
                gath[b] = gather(nxt, b)
            else:
                store[b].wait()

    return k(table, idx_flat)


TCOLS = 2048


def _tc_transpose(wt_t):
    n = wt_t.shape[1]

    def body(x_ref, o_ref):
        x = x_ref[...]
        o_ref[...] = (
            x.reshape(32, TCOLS // 4, 4).transpose(1, 2, 0).reshape(TCOLS // 4, 128)
        )

    return pl.pallas_call(
        body,
        grid=(pl.cdiv(n, TCOLS),),
        in_specs=[pl.BlockSpec((32, TCOLS), lambda i: (0, i))],
        out_specs=pl.BlockSpec((TCOLS // 4, 128), lambda i: (i, 0)),
        out_shape=jax.ShapeDtypeStruct((n * 32 // 128, 128), jnp.float32),
    )(wt_t)


def kernel(L_self_modules_embedding_parameters_weight_, L_batch_):
    wt_t = jnp.transpose(L_self_modules_embedding_parameters_weight_)
    table = _tc_transpose(wt_t).reshape(-1, D)
    idx = L_batch_.reshape(-1).astype(jnp.int32)
    out = _sc_gather(table, idx)
    return (out.reshape(L_batch_.shape + (D,)),)


# speedup vs baseline: 3.2198x; 3.2198x over previous
import functools

import jax
import jax.numpy as jnp
from jax import lax
from jax.experimental import pallas as pl
from jax.experimental.pallas import tpu as pltpu
from jax.experimental.pallas import tpu_sc as plsc

D = 32
NB = 16384
NS = 26
B = NB * NS
SH = 13
HROWS = 128 * SH


def _sc_gather(table, idx_flat):
    info = plsc.get_sparse_core_info()
    nc = info.num_cores
    nw = nc * info.num_subcores
    b_per_w = B // nw
    mesh = plsc.VectorSubcoreMesh(core_axis_name="c", subcore_axis_name="s")

    @functools.partial(
        pl.kernel,
        mesh=mesh,
        out_type=jax.ShapeDtypeStruct((NS * 4, NB // 128, 8, 128), jnp.float32),
        scratch_types=[
            pltpu.VMEM((3328,), jnp.int32),
            pltpu.VMEM((2 * HROWS,), jnp.int32),
            pltpu.VMEM((2 * HROWS, D), jnp.float32),
            [pltpu.VMEM((D, 128), jnp.float32) for _ in range(2)],
            [pltpu.SemaphoreType.DMA for _ in range(2)],
            [pltpu.SemaphoreType.DMA for _ in range(2)],
        ],
        compiler_params=pltpu.CompilerParams(
            use_tc_tiling_on_sc=False, needs_layout_passes=False
        ),
    )
    def k(table_hbm, idx_hbm, out_hbm, idx_v, cidx, dest, staged, gsem, ssem):
        wid = lax.axis_index("s") * nc + lax.axis_index("c")
        base = wid * b_per_w
        lanes = lax.iota(jnp.int32, 16)

        def build_and_fire(jj, h):
            def sbody(si, carry):
                s = h * SH + si
                for g in range(8):
                    pos = lanes * NS + (16 * g * NS + s)
                    v = plsc.load_gather(idx_v, [pos])
                    cidx[pl.ds(h * HROWS + si * 128 + 16 * g, 16)] = v
                return carry

            lax.fori_loop(0, SH, sbody, 0)
            pltpu.async_copy(
                table_hbm.at[cidx.at[pl.ds(h * HROWS, HROWS)]],
                dest.at[pl.ds(h * HROWS, HROWS)],
                gsem[h],
            )

        def wait_gather(h):
            pltpu.make_async_copy(
                table_hbm.at[cidx.at[pl.ds(h * HROWS, HROWS)]],
                dest.at[pl.ds(h * HROWS, HROWS)],
                gsem[h],
            ).wait()

        def drain_panel(p):
            for d0 in range(4):
                pltpu.make_async_copy(
                    staged[p].at[pl.ds(8 * d0, 8)], out_hbm.at[0, 0], ssem[p]
                ).wait()

        def process(jj, h, first):
            j = 4 * wid + jj
            for si in range(SH):
                p = si % 2
                if si >= 2:
                    drain_panel(p)
                elif not first:
                    drain_panel(p)
                elif first:
                    @pl.when(jj >= 1)
                    def _():
                        drain_panel(p)

                rowbase = h * HROWS + si * 128

                def dbody(d, carry):
                    cols = jnp.full((16,), d, jnp.int32)
                    for g in range(8):
                        rows = rowbase + 16 * g + lanes
                        v = plsc.load_gather(dest, [rows, cols])
                        staged[p][d, pl.ds(16 * g, 16)] = v
                    return carry

                lax.fori_loop(0, D, dbody, 0)
                sd = (h * SH + si) * 4
                for d0 in range(4):
                    pltpu.async_copy(
                        staged[p].at[pl.ds(8 * d0, 8)],
                        out_hbm.at[sd + d0, j],
                        ssem[p],
                    )

        def jbody(jj, carry):
            pltpu.sync_copy(idx_hbm.at[pl.ds(base + jj * 3328, 3328)], idx_v)
            build_and_fire(jj, 0)
            build_and_fire(jj, 1)
            wait_gather(0)
            process(jj, 0, True)
            wait_gather(1)
            process(jj, 1, False)
            return carry

        lax.fori_loop(0, 4, jbody, 0)
        drain_panel(0)
        drain_panel(1)

    return k(table, idx_flat)


def kernel(L_self_modules_embedding_parameters_weight_, L_batch_):
    table = L_self_modules_embedding_parameters_weight_
    idx = L_batch_.reshape(-1).astype(jnp.int32)
    flat = _sc_gather(table, idx).reshape(-1)
    x5 = flat.reshape(NS, 4, NB // 128, 8, 128)
    x = x5.transpose(0, 1, 3, 2, 4).reshape(NS, D, NB)
    return (x.transpose(2, 0, 1),)


# speedup vs baseline: 4.5792x; 1.4222x over previous
import functools

import jax
import jax.numpy as jnp
from jax import lax
from jax.experimental import pallas as pl
from jax.experimental.pallas import tpu as pltpu
from jax.experimental.pallas import tpu_sc as plsc

D = 32
NB = 16384
NS = 26
B = NB * NS
SH = 13
HROWS = 128 * SH


def _sc_gather(table, idx_flat):
    info = plsc.get_sparse_core_info()
    nc = info.num_cores
    nw = nc * info.num_subcores
    b_per_w = B // nw
    mesh = plsc.VectorSubcoreMesh(core_axis_name="c", subcore_axis_name="s")

    @functools.partial(
        pl.kernel,
        mesh=mesh,
        out_type=jax.ShapeDtypeStruct((NS * 4, NB // 128, 8, 128), jnp.float32),
        scratch_types=[
            pltpu.VMEM((3328,), jnp.int32),
            pltpu.VMEM((2 * HROWS,), jnp.int32),
            pltpu.VMEM((2 * HROWS, D), jnp.float32),
            [pltpu.VMEM((D, 131), jnp.float32) for _ in range(4)],
            [pltpu.SemaphoreType.DMA for _ in range(2)],
            [pltpu.SemaphoreType.DMA for _ in range(4)],
        ],
        compiler_params=pltpu.CompilerParams(
            use_tc_tiling_on_sc=False, needs_layout_passes=False
        ),
    )
    def k(table_hbm, idx_hbm, out_hbm, idx_v, cidx, dest, staged, gsem, ssem):
        wid = lax.axis_index("s") * nc + lax.axis_index("c")
        base = wid * b_per_w
        lanes = lax.iota(jnp.int32, 16)

        def build_and_fire(jj, h):
            def sbody(si, carry):
                s = h * SH + si
                for g in range(8):
                    pos = lanes * NS + (16 * g * NS + s)
                    v = plsc.load_gather(idx_v, [pos])
                    cidx[pl.ds(h * HROWS + si * 128 + 16 * g, 16)] = v
                return carry

            lax.fori_loop(0, SH, sbody, 0)
            pltpu.async_copy(
                table_hbm.at[cidx.at[pl.ds(h * HROWS, HROWS)]],
                dest.at[pl.ds(h * HROWS, HROWS)],
                gsem[h],
            )

        def wait_gather(h):
            pltpu.make_async_copy(
                table_hbm.at[cidx.at[pl.ds(h * HROWS, HROWS)]],
                dest.at[pl.ds(h * HROWS, HROWS)],
                gsem[h],
            ).wait()

        def drain_panel(p):
            for d0 in range(4):
                pltpu.make_async_copy(
                    staged[p].at[pl.ds(8 * d0, 8), pl.ds(0, 128)],
                    out_hbm.at[0, 0],
                    ssem[p],
                ).wait()

        def process(jj, h, first):
            j = 4 * wid + jj
            for si in range(SH):
                p = si % 4
                if si >= 4:
                    drain_panel(p)
                elif not first:
                    drain_panel(p)
                elif first:
                    @pl.when(jj >= 1)
                    def _():
                        drain_panel(p)

                rowbase = h * HROWS + si * 128

                def pbody(bg, carry):
                    for u in range(8):
                        br = bg * 8 + u
                        row = rowbase + br
                        v0 = dest[row, pl.ds(0, 16)]
                        v1 = dest[row, pl.ds(16, 16)]
                        brv = lanes * 0 + br
                        plsc.store_scatter(staged[p], [lanes, brv], v0)
                        plsc.store_scatter(staged[p], [lanes + 16, brv], v1)
                    return carry

                lax.fori_loop(0, 16, pbody, 0)
                sd = (h * SH + si) * 4
                for d0 in range(4):
                    pltpu.async_copy(
                        staged[p].at[pl.ds(8 * d0, 8), pl.ds(0, 128)],
                        out_hbm.at[sd + d0, j],
                        ssem[p],
                    )

        def jbody(jj, carry):
            pltpu.sync_copy(idx_hbm.at[pl.ds(base + jj * 3328, 3328)], idx_v)
            build_and_fire(jj, 0)
            build_and_fire(jj, 1)
            wait_gather(0)
            process(jj, 0, True)
            wait_gather(1)
            process(jj, 1, False)
            return carry

        lax.fori_loop(0, 4, jbody, 0)
        for p in range(4):
            drain_panel(p)

    return k(table, idx_flat)


def kernel(L_self_modules_embedding_parameters_weight_, L_batch_):
    table = L_self_modules_embedding_parameters_weight_
    idx = L_batch_.reshape(-1).astype(jnp.int32)
    flat = _sc_gather(table, idx).reshape(-1)
    x5 = flat.reshape(NS, 4, NB // 128, 8, 128)
    x = x5.transpose(0, 1, 3, 2, 4).reshape(NS, D, NB)
    return (x.transpose(2, 0, 1),)


# speedup vs baseline: 4.5822x; 1.0007x over previous
import functools

import jax
import jax.numpy as jnp
from jax import lax
from jax.experimental import pallas as pl
from jax.experimental.pallas import tpu as pltpu
from jax.experimental.pallas import tpu_sc as plsc

D = 32
NB = 16384
NS = 26
B = NB * NS
SH = 13
HROWS = 128 * SH


def _sc_gather(table, idx_flat):
    info = plsc.get_sparse_core_info()
    nc = info.num_cores
    nw = nc * info.num_subcores
    b_per_w = B // nw
    mesh = plsc.VectorSubcoreMesh(core_axis_name="c", subcore_axis_name="s")

    @functools.partial(
        pl.kernel,
        mesh=mesh,
        out_type=jax.ShapeDtypeStruct((NS * 4, NB // 128, 8, 128), jnp.float32),
        scratch_types=[
            pltpu.VMEM((3328,), jnp.int32),
            pltpu.VMEM((2 * HROWS,), jnp.int32),
            pltpu.VMEM((2 * HROWS, D), jnp.float32),
            [pltpu.VMEM((D, 131), jnp.float32) for _ in range(4)],
            [pltpu.SemaphoreType.DMA for _ in range(2)],
            [pltpu.SemaphoreType.DMA for _ in range(4)],
        ],
        compiler_params=pltpu.CompilerParams(
            use_tc_tiling_on_sc=False, needs_layout_passes=False
        ),
    )
    def k(table_hbm, idx_hbm, out_hbm, idx_v, cidx, dest, staged, gsem, ssem):
        wid = lax.axis_index("s") * nc + lax.axis_index("c")
        base = wid * b_per_w
        lanes = lax.iota(jnp.int32, 16)

        def build_and_fire(jj, h):
            def sbody(si, carry):
                s = h * SH + si
                for g in range(8):
                    pos = lanes * NS + (16 * g * NS + s)
                    v = plsc.load_gather(idx_v, [pos])
                    cidx[pl.ds(h * HROWS + si * 128 + 16 * g, 16)] = v
                return carry

            lax.fori_loop(0, SH, sbody, 0)
            pltpu.async_copy(
                table_hbm.at[cidx.at[pl.ds(h * HROWS, HROWS)]],
                dest.at[pl.ds(h * HROWS, HROWS)],
                gsem[h],
            )

        def wait_gather(h):
            pltpu.make_async_copy(
                table_hbm.at[cidx.at[pl.ds(h * HROWS, HROWS)]],
                dest.at[pl.ds(h * HROWS, HROWS)],
                gsem[h],
            ).wait()

        def drain_panel(p):
            for d0 in range(4):
                pltpu.make_async_copy(
                    staged[p].at[pl.ds(8 * d0, 8), pl.ds(0, 128)],
                    out_hbm.at[0, 0],
                    ssem[p],
                ).wait()

        def process(jj, h, first):
            j = 4 * wid + jj
            for si in range(SH):
                p = si % 4
                if si >= 4:
                    drain_panel(p)
                elif not first:
                    drain_panel(p)
                elif first:
                    @pl.when(jj >= 1)
                    def _():
                        drain_panel(p)

                rowbase = h * HROWS + si * 128

                def pbody(bg, carry):
                    for u in range(8):
                        br = bg * 8 + u
                        row = rowbase + br
                        v0 = dest[row, pl.ds(0, 16)]
                        v1 = dest[row, pl.ds(16, 16)]
                        brv = lanes * 0 + br
                        plsc.store_scatter(staged[p], [lanes, brv], v0)
                        plsc.store_scatter(staged[p], [lanes + 16, brv], v1)
                    return carry

                lax.fori_loop(0, 16, pbody, 0)
                sd = (h * SH + si) * 4
                for d0 in range(4):
                    pltpu.async_copy(
                        staged[p].at[pl.ds(8 * d0, 8), pl.ds(0, 128)],
                        out_hbm.at[sd + d0, j],
                        ssem[p],
                    )

        def jbody(jj, carry):
            pltpu.sync_copy(idx_hbm.at[pl.ds(base + jj * 3328, 3328)], idx_v)
            build_and_fire(jj, 0)
            build_and_fire(jj, 1)
            wait_gather(0)
            process(jj, 0, True)
            wait_gather(1)
            process(jj, 1, False)
            return carry

        lax.fori_loop(0, 4, jbody, 0)
        for p in range(4):
            drain_panel(p)

    return k(table, idx_flat)


def kernel(L_self_modules_embedding_parameters_weight_, L_batch_):
    t128 = L_self_modules_embedding_parameters_weight_.reshape(250000, 128)
    t128 = jax.lax.optimization_barrier(t128)
    table = t128.reshape(1000000, 32)
    idx = L_batch_.reshape(-1).astype(jnp.int32)
    flat = _sc_gather(table, idx).reshape(-1)
    x5 = flat.reshape(NS, 4, NB // 128, 8, 128)
    x = x5.transpose(0, 1, 3, 2, 4).reshape(NS, D, NB)
    return (x.transpose(2, 0, 1),)
